# speedup vs baseline: 1.5217x; 1.5217x over previous
import functools

import jax
import jax.numpy as jnp
from jax import lax
from jax.experimental import pallas as pl
from jax.experimental.pallas import tpu as pltpu
from jax.experimental.pallas import tpu_sc as plsc

_NC = 2
_NS = 16
_L = 16
_NW = _NC * _NS

_CHUNK = 32
_NBUF = 3


def _sc_body(n_rows, d, table_hbm, slm1_hbm, out_hbm, slm1_v, idx_v,
             bufs_v, *sems):
    b_per_w = n_rows // _NW
    n_chunks = b_per_w // _CHUNK
    wid = lax.axis_index("s") * _NC + lax.axis_index("c")
    base = wid * b_per_w

    pltpu.sync_copy(slm1_hbm, slm1_v)
    slm1 = slm1_v[...]

    for i in range(b_per_w // _L):
        rows = lax.broadcasted_iota(jnp.int32, (_L,), 0) + (base + i * _L)
        idx_v[pl.ds(i * _L, _L)] = jnp.minimum(rows, slm1)

    def gather(c):
        return pltpu.async_copy(
            table_hbm.at[idx_v.at[pl.ds(c * _CHUNK, _CHUNK)]],
            bufs_v.at[c % _NBUF],
            sems[c % _NBUF],
        )

    inflight = [gather(c) for c in range(min(_NBUF, n_chunks))]
    for c in range(n_chunks):
        inflight[c % _NBUF].wait()
        pltpu.sync_copy(
            bufs_v.at[c % _NBUF],
            out_hbm.at[pl.ds(base + c * _CHUNK, _CHUNK)],
        )
        if c + _NBUF < n_chunks:
            inflight[c % _NBUF] = gather(c + _NBUF)


@functools.partial(jax.jit, static_argnums=(1, 2))
def _lookup(table, n_rows, d, slm1):
    body = functools.partial(_sc_body, n_rows, d)
    return pl.kernel(
        body,
        out_type=jax.ShapeDtypeStruct((n_rows, d), table.dtype),
        mesh=plsc.VectorSubcoreMesh(core_axis_name="c", subcore_axis_name="s"),
        scratch_types=[
            pltpu.VMEM((_L,), jnp.int32),
            pltpu.VMEM((n_rows // _NW,), jnp.int32),
            pltpu.VMEM((_NBUF, _CHUNK, d), table.dtype),
        ] + [pltpu.SemaphoreType.DMA] * _NBUF,
    )(table, slm1)


def kernel(position_embeddings, seq_len):
    n_rows, d = position_embeddings.shape
    slm1 = jnp.full((_L,), jnp.asarray(seq_len, jnp.int32) - 1, jnp.int32)
    return _lookup(position_embeddings, n_rows, d, slm1)
